# baseline (speedup 1.0000x reference)
import functools

import jax
import jax.numpy as jnp
from jax import lax
from jax.experimental import pallas as pl
from jax.experimental.pallas import tpu as pltpu
from jax.experimental.pallas import tpu_sc as plsc

_K_RATIO = 0.1
_MIN_K = 5

_NC = 2
_NS = 16
_NW = _NC * _NS
_L = 16
_BITS = (10, 8, 8, 6)
_SHIFTS = (22, 14, 6, 0)
_NB1 = 1 << _BITS[0]
_HS = _NB1 + 1
_CAP = 2049


def _f32_to_key(x):
    bu = lax.bitcast_convert_type(x, jnp.uint32)
    sign = bu >> jnp.uint32(31)
    return bu ^ ((sign * jnp.uint32(0xFFFFFFFF)) | jnp.uint32(0x80000000))


def _key_to_f32(key):
    high = key >= jnp.uint32(0x80000000)
    bu = jnp.where(high, key ^ jnp.uint32(0x80000000), ~key)
    return lax.bitcast_convert_type(bu, jnp.float32)


def _sc_body(x_hbm, out_hbm, xrow, bufa, bufb, hist, red, red2, red3, resv,
             dsem, *, n, k, rows_per_w):
    cid = lax.axis_index("c")
    sid = lax.axis_index("s")
    wid = cid * _NS + sid
    lane = lax.iota(jnp.int32, _L)
    lane_base = lane * _HS
    ones = jnp.ones((_L,), jnp.int32)
    kf = jnp.float32(k)

    def zero_hist():
        def body(j, _):
            hist[pl.ds(j * _L, _L)] = jnp.zeros((_L,), jnp.int32)
            return 0
        lax.fori_loop(0, _HS * _L // _L, body, 0, unroll=8)

    def scan_level(r_needed, nb):
        nch = nb // _L
        @plsc.parallel_loop(0, nch, unroll=2)
        def _(j):
            zero = jnp.zeros((_L,), jnp.int32)
            acc = hist[pl.ds(j * _L, _L)]
            hist[pl.ds(j * _L, _L)] = zero
            for l in range(1, _L):
                acc = acc + hist[pl.ds(l * _HS + j * _L, _L)]
                hist[pl.ds(l * _HS + j * _L, _L)] = zero
            revacc = lax.rev(acc, (0,))
            red[pl.ds(j * _L, _L)] = lax.cumsum(revacc)
            red3[pl.ds(j * _L, _L)] = revacc

        carry_above = jnp.int32(0)
        for c in range(nch // _L + (1 if nch % _L else 0) - 1, -1, -1):
            tot = plsc.load_gather(red, [(lane + c * _L) * _L + (_L - 1)])
            tot = jnp.where((lane + c * _L) < nch, tot, 0)
            suf = lax.rev(lax.cumsum(lax.rev(tot, (0,))), (0,)) - tot
            red2[pl.ds(c * _L, _L)] = suf + carry_above
            carry_above = carry_above + jnp.sum(tot)

        init = (jnp.full((_L,), -1, jnp.int32),
                jnp.full((_L,), jnp.iinfo(jnp.int32).max, jnp.int32))

        @plsc.parallel_loop(0, nch, unroll=2, carry=init)
        def find(j, carry):
            bbest, cmin = carry
            racc = plsc.load_gather(red2, [jnp.zeros((_L,), jnp.int32) + j])
            cge = red[pl.ds(j * _L, _L)] + racc
            cgt = cge - red3[pl.ds(j * _L, _L)]
            mask = cge >= r_needed
            bvec = j * _L + (_L - 1) - lane
            bbest = jnp.maximum(bbest, jnp.where(mask, bvec, -1))
            cmin = jnp.minimum(cmin, jnp.where(mask, cgt,
                                               jnp.iinfo(jnp.int32).max))
            return bbest, cmin
        bbest_v, cmin_v = find
        return jnp.max(bbest_v), jnp.min(cmin_v)

    lane_cap = lane * _CAP

    def scatter_level(src, cnt_v, trips, shift, bmask):
        @plsc.parallel_loop(0, trips, unroll=4)
        def _(t):
            key = lax.bitcast_convert_type(
                plsc.load_gather(src, [lane_cap + t]), jnp.uint32)
            valid = cnt_v > t
            b = ((key >> jnp.uint32(shift))
                 & jnp.uint32(bmask)).astype(jnp.int32)
            plsc.addupdate_scatter(hist, [lane_base + b], ones, mask=valid)

    def partition(src, dst, cnt_v, bb, shift, bmask, s_acc, *, trips=None,
                  from_f32=False, unroll=1):
        def body(i, carry):
            cnt, s = carry
            if from_f32:
                x = src[pl.ds(i * _L, _L)]
                key = _f32_to_key(x)
                b = (key >> jnp.uint32(shift)).astype(jnp.int32)
                mgt = b > bb
                meq = b == bb
            else:
                key = lax.bitcast_convert_type(
                    plsc.load_gather(src, [lane_cap + i]), jnp.uint32)
                x = _key_to_f32(key)
                valid = cnt_v > i
                b = ((key >> jnp.uint32(shift))
                     & jnp.uint32(bmask)).astype(jnp.int32)
                mgt = valid & (b > bb)
                meq = valid & (b == bb)
            s = s + jnp.where(mgt, x, jnp.float32(0.0))
            plsc.store_scatter(dst, [lane_cap + cnt],
                               lax.bitcast_convert_type(key, jnp.int32),
                               mask=meq)
            return cnt + meq.astype(jnp.int32), s
        new_cnt, s_acc = plsc.parallel_loop(
            0, trips, unroll=unroll,
            carry=(jnp.zeros((_L,), jnp.int32), s_acc))(
                lambda i, carry: body(i, carry))
        return new_cnt, s_acc

    n_chunks = 4
    ch = n // n_chunks

    def row_body(rloc, res_acc):
        row = wid * rows_per_w + rloc
        copies = [pltpu.async_copy(x_hbm.at[row, pl.ds(c * ch, ch)],
                                   xrow.at[pl.ds(c * ch, ch)], dsem.at[c])
                  for c in range(n_chunks)]
        for c in range(n_chunks):
            copies[c].wait()

            @plsc.parallel_loop(0, ch // _L, unroll=8)
            def _(i, _c=c):
                x = xrow[pl.ds(_c * ch + i * _L, _L)]
                key = _f32_to_key(x)
                b = (key >> jnp.uint32(_SHIFTS[0])).astype(jnp.int32)
                plsc.addupdate_scatter(hist, [lane_base + b], ones)

        b1, c1 = scan_level(jnp.int32(k), _NB1)
        cnt1, s = partition(xrow, bufb, None, b1, _SHIFTS[0], 0,
                            jnp.zeros((_L,), jnp.float32),
                            trips=n // _L, from_f32=True, unroll=8)
        a = c1

        t1 = jnp.max(cnt1)
        m2 = (1 << _BITS[1]) - 1
        scatter_level(bufb, cnt1, t1, _SHIFTS[1], m2)
        b2, c2 = scan_level(k - a, 1 << _BITS[1])
        cnt2, s = partition(bufb, bufa, cnt1, b2, _SHIFTS[1], m2, s, trips=t1,
                            unroll=2)
        a = a + c2

        t2 = jnp.max(cnt2)
        m3 = (1 << _BITS[2]) - 1
        scatter_level(bufa, cnt2, t2, _SHIFTS[2], m3)
        b3, c3 = scan_level(k - a, 1 << _BITS[2])
        cnt3, s = partition(bufa, bufb, cnt2, b3, _SHIFTS[2], m3, s, trips=t2)
        a = a + c3

        t3 = jnp.max(cnt3)
        m4 = (1 << _BITS[3]) - 1
        scatter_level(bufb, cnt3, t3, _SHIFTS[3], m4)
        b4, c4 = scan_level(k - a, 1 << _BITS[3])
        _, s = partition(bufb, bufa, cnt3, b4, _SHIFTS[3], m4, s, trips=t3)
        a = a + c4

        tu = ((b1.astype(jnp.uint32) << jnp.uint32(_SHIFTS[0]))
              | (b2.astype(jnp.uint32) << jnp.uint32(_SHIFTS[1]))
              | (b3.astype(jnp.uint32) << jnp.uint32(_SHIFTS[2]))
              | b4.astype(jnp.uint32))
        tx = _key_to_f32(jnp.zeros((_L,), jnp.uint32) + tu)
        r_v = (jnp.full((_L,), k, jnp.int32) - a).astype(jnp.float32)
        s_tot = jnp.zeros((_L,), jnp.float32) + jnp.sum(s)
        val_v = (s_tot + r_v * tx) / kf
        return jnp.where(lane == rloc, val_v, res_acc)

    zero_hist()
    res = lax.fori_loop(0, rows_per_w, row_body, jnp.zeros((_L,), jnp.float32))
    resv[...] = res
    pltpu.sync_copy(resv, out_hbm.at[pl.ds(wid * _L, _L)])


def kernel(patch_logits):
    if patch_logits.ndim == 4:
        b = patch_logits.shape[0]
        patch_logits = patch_logits.reshape(b, -1)
    rows, n = patch_logits.shape
    k = max(_MIN_K, int(-(-n * _K_RATIO // 1)))
    rows_per_w = rows // _NW
    mesh = plsc.VectorSubcoreMesh(core_axis_name="c", subcore_axis_name="s",
                                  num_cores=_NC, num_subcores=_NS)
    body = functools.partial(_sc_body, n=n, k=k, rows_per_w=rows_per_w)
    out = pl.kernel(
        body,
        out_type=jax.ShapeDtypeStruct((_NW * _L,), jnp.float32),
        mesh=mesh,
        compiler_params=pltpu.CompilerParams(needs_layout_passes=False),
        scratch_types=[
            pltpu.VMEM((n,), jnp.float32),
            pltpu.VMEM((_CAP * _L,), jnp.int32),
            pltpu.VMEM((_CAP * _L,), jnp.int32),
            pltpu.VMEM((_HS * _L,), jnp.int32),
            pltpu.VMEM((_NB1,), jnp.int32),
            pltpu.VMEM((_NB1 // _L,), jnp.int32),
            pltpu.VMEM((_NB1,), jnp.int32),
            pltpu.VMEM((_L,), jnp.float32),
            pltpu.SemaphoreType.DMA((4,)),
        ],
    )(patch_logits)
    return out.reshape(_NW, _L)[:, :rows_per_w].reshape(rows, 1)


# speedup vs baseline: 1.0062x; 1.0062x over previous
import functools

import jax
import jax.numpy as jnp
from jax import lax
from jax.experimental import pallas as pl
from jax.experimental.pallas import tpu as pltpu
from jax.experimental.pallas import tpu_sc as plsc

_K_RATIO = 0.1
_MIN_K = 5

_NC = 2
_NS = 16
_NW = _NC * _NS
_L = 16
_BITS = (10, 8, 8, 6)
_SHIFTS = (22, 14, 6, 0)
_NB1 = 1 << _BITS[0]
_HS = _NB1 + 1
_CAP = 2049


def _f32_to_key(x):
    bu = lax.bitcast_convert_type(x, jnp.uint32)
    sign = bu >> jnp.uint32(31)
    return bu ^ ((sign * jnp.uint32(0xFFFFFFFF)) | jnp.uint32(0x80000000))


def _key_to_f32(key):
    high = key >= jnp.uint32(0x80000000)
    bu = jnp.where(high, key ^ jnp.uint32(0x80000000), ~key)
    return lax.bitcast_convert_type(bu, jnp.float32)


def _sc_body(x_hbm, out_hbm, xrow, bufa, bufb, hist, red, red2, red3, resv,
             *, n, k, rows_per_w):
    cid = lax.axis_index("c")
    sid = lax.axis_index("s")
    wid = cid * _NS + sid
    lane = lax.iota(jnp.int32, _L)
    lane_base = lane * _HS
    ones = jnp.ones((_L,), jnp.int32)
    kf = jnp.float32(k)

    def zero_hist():
        def body(j, _):
            hist[pl.ds(j * _L, _L)] = jnp.zeros((_L,), jnp.int32)
            return 0
        lax.fori_loop(0, _HS * _L // _L, body, 0, unroll=8)

    def scan_level(r_needed, nb):
        nch = nb // _L
        @plsc.parallel_loop(0, nch, unroll=2)
        def _(j):
            zero = jnp.zeros((_L,), jnp.int32)
            acc = hist[pl.ds(j * _L, _L)]
            hist[pl.ds(j * _L, _L)] = zero
            for l in range(1, _L):
                acc = acc + hist[pl.ds(l * _HS + j * _L, _L)]
                hist[pl.ds(l * _HS + j * _L, _L)] = zero
            revacc = lax.rev(acc, (0,))
            red[pl.ds(j * _L, _L)] = lax.cumsum(revacc)
            red3[pl.ds(j * _L, _L)] = revacc

        carry_above = jnp.int32(0)
        for c in range(nch // _L + (1 if nch % _L else 0) - 1, -1, -1):
            tot = plsc.load_gather(red, [(lane + c * _L) * _L + (_L - 1)])
            tot = jnp.where((lane + c * _L) < nch, tot, 0)
            suf = lax.rev(lax.cumsum(lax.rev(tot, (0,))), (0,)) - tot
            red2[pl.ds(c * _L, _L)] = suf + carry_above
            carry_above = carry_above + jnp.sum(tot)

        init = (jnp.full((_L,), -1, jnp.int32),
                jnp.full((_L,), jnp.iinfo(jnp.int32).max, jnp.int32))

        @plsc.parallel_loop(0, nch, unroll=2, carry=init)
        def find(j, carry):
            bbest, cmin = carry
            racc = plsc.load_gather(red2, [jnp.zeros((_L,), jnp.int32) + j])
            cge = red[pl.ds(j * _L, _L)] + racc
            cgt = cge - red3[pl.ds(j * _L, _L)]
            mask = cge >= r_needed
            bvec = j * _L + (_L - 1) - lane
            bbest = jnp.maximum(bbest, jnp.where(mask, bvec, -1))
            cmin = jnp.minimum(cmin, jnp.where(mask, cgt,
                                               jnp.iinfo(jnp.int32).max))
            return bbest, cmin
        bbest_v, cmin_v = find
        return jnp.max(bbest_v), jnp.min(cmin_v)

    lane_cap = lane * _CAP

    def scatter_level(src, cnt_v, trips, shift, bmask):
        @plsc.parallel_loop(0, trips, unroll=4)
        def _(t):
            key = lax.bitcast_convert_type(
                plsc.load_gather(src, [lane_cap + t]), jnp.uint32)
            valid = cnt_v > t
            b = ((key >> jnp.uint32(shift))
                 & jnp.uint32(bmask)).astype(jnp.int32)
            plsc.addupdate_scatter(hist, [lane_base + b], ones, mask=valid)

    def partition(src, dst, cnt_v, bb, shift, bmask, s_acc, *, trips=None,
                  from_f32=False, unroll=1):
        def body(i, carry):
            cnt, s = carry
            if from_f32:
                x = src[pl.ds(i * _L, _L)]
                key = _f32_to_key(x)
                b = (key >> jnp.uint32(shift)).astype(jnp.int32)
                mgt = b > bb
                meq = b == bb
            else:
                key = lax.bitcast_convert_type(
                    plsc.load_gather(src, [lane_cap + i]), jnp.uint32)
                x = _key_to_f32(key)
                valid = cnt_v > i
                b = ((key >> jnp.uint32(shift))
                     & jnp.uint32(bmask)).astype(jnp.int32)
                mgt = valid & (b > bb)
                meq = valid & (b == bb)
            s = s + jnp.where(mgt, x, jnp.float32(0.0))
            plsc.store_scatter(dst, [lane_cap + cnt],
                               lax.bitcast_convert_type(key, jnp.int32),
                               mask=meq)
            return cnt + meq.astype(jnp.int32), s
        new_cnt, s_acc = plsc.parallel_loop(
            0, trips, unroll=unroll,
            carry=(jnp.zeros((_L,), jnp.int32), s_acc))(
                lambda i, carry: body(i, carry))
        return new_cnt, s_acc

    def row_body(rloc, res_acc):
        pltpu.sync_copy(x_hbm.at[wid * rows_per_w + rloc], xrow)

        @plsc.parallel_loop(0, n // _L, unroll=8)
        def _(i):
            x = xrow[pl.ds(i * _L, _L)]
            key = _f32_to_key(x)
            b = (key >> jnp.uint32(_SHIFTS[0])).astype(jnp.int32)
            plsc.addupdate_scatter(hist, [lane_base + b], ones)

        b1, c1 = scan_level(jnp.int32(k), _NB1)
        cnt1, s = partition(xrow, bufb, None, b1, _SHIFTS[0], 0,
                            jnp.zeros((_L,), jnp.float32),
                            trips=n // _L, from_f32=True, unroll=8)
        a = c1

        t1 = jnp.max(cnt1)
        m2 = (1 << _BITS[1]) - 1
        scatter_level(bufb, cnt1, t1, _SHIFTS[1], m2)
        b2, c2 = scan_level(k - a, 1 << _BITS[1])
        cnt2, s = partition(bufb, bufa, cnt1, b2, _SHIFTS[1], m2, s, trips=t1,
                            unroll=2)
        a = a + c2

        t2 = jnp.max(cnt2)
        m3 = (1 << _BITS[2]) - 1
        scatter_level(bufa, cnt2, t2, _SHIFTS[2], m3)
        b3, c3 = scan_level(k - a, 1 << _BITS[2])
        cnt3, s = partition(bufa, bufb, cnt2, b3, _SHIFTS[2], m3, s, trips=t2)
        a = a + c3

        t3 = jnp.max(cnt3)
        m4 = (1 << _BITS[3]) - 1
        scatter_level(bufb, cnt3, t3, _SHIFTS[3], m4)
        b4, c4 = scan_level(k - a, 1 << _BITS[3])
        _, s = partition(bufb, bufa, cnt3, b4, _SHIFTS[3], m4, s, trips=t3)
        a = a + c4

        tu = ((b1.astype(jnp.uint32) << jnp.uint32(_SHIFTS[0]))
              | (b2.astype(jnp.uint32) << jnp.uint32(_SHIFTS[1]))
              | (b3.astype(jnp.uint32) << jnp.uint32(_SHIFTS[2]))
              | b4.astype(jnp.uint32))
        tx = _key_to_f32(jnp.zeros((_L,), jnp.uint32) + tu)
        r_v = (jnp.full((_L,), k, jnp.int32) - a).astype(jnp.float32)
        s_tot = jnp.zeros((_L,), jnp.float32) + jnp.sum(s)
        val_v = (s_tot + r_v * tx) / kf
        return jnp.where(lane == rloc, val_v, res_acc)

    zero_hist()
    res = lax.fori_loop(0, rows_per_w, row_body, jnp.zeros((_L,), jnp.float32))
    resv[...] = res
    pltpu.sync_copy(resv, out_hbm.at[pl.ds(wid * _L, _L)])


def kernel(patch_logits):
    if patch_logits.ndim == 4:
        b = patch_logits.shape[0]
        patch_logits = patch_logits.reshape(b, -1)
    rows, n = patch_logits.shape
    k = max(_MIN_K, int(-(-n * _K_RATIO // 1)))
    rows_per_w = rows // _NW
    mesh = plsc.VectorSubcoreMesh(core_axis_name="c", subcore_axis_name="s",
                                  num_cores=_NC, num_subcores=_NS)
    body = functools.partial(_sc_body, n=n, k=k, rows_per_w=rows_per_w)
    out = pl.kernel(
        body,
        out_type=jax.ShapeDtypeStruct((_NW * _L,), jnp.float32),
        mesh=mesh,
        compiler_params=pltpu.CompilerParams(needs_layout_passes=False),
        scratch_types=[
            pltpu.VMEM((n,), jnp.float32),
            pltpu.VMEM((_CAP * _L,), jnp.int32),
            pltpu.VMEM((_CAP * _L,), jnp.int32),
            pltpu.VMEM((_HS * _L,), jnp.int32),
            pltpu.VMEM((_NB1,), jnp.int32),
            pltpu.VMEM((_NB1 // _L,), jnp.int32),
            pltpu.VMEM((_NB1,), jnp.int32),
            pltpu.VMEM((_L,), jnp.float32),
        ],
    )(patch_logits)
    return out.reshape(_NW, _L)[:, :rows_per_w].reshape(rows, 1)
